# Initial kernel scaffold; baseline (speedup 1.0000x reference)
import jax
import jax.numpy as jnp
from jax.experimental import pallas as pl


def kernel(z0, z1, src, dst, r, r_hat, edges, natoms0, natoms1, W_emb, b_emb, Ws2d, Wd2s, W_ro, b_ro):
    raise NotImplementedError("write your pallas kernel here")


# speedup vs baseline: 2.3442x; 2.3442x over previous
import functools

import numpy as np
import jax
import jax.numpy as jnp
from jax import lax
from jax.experimental import pallas as pl
from jax.experimental.pallas import tpu as pltpu
from jax.experimental.pallas import tpu_sc as plsc

NC, NS = 2, 16
NW = NC * NS
CB = 128
NF = 6
BE = 640
BR = 1000


def _sc_gather(n_chunks, dim):
    T = -(-n_chunks // NW)
    mesh = plsc.VectorSubcoreMesh(core_axis_name="c", subcore_axis_name="s")

    @functools.partial(
        pl.kernel,
        out_type=jax.ShapeDtypeStruct((n_chunks * CB, dim), jnp.float32),
        mesh=mesh,
        scratch_types=[
            pltpu.VMEM((CB,), jnp.int32),
            pltpu.VMEM((CB, dim), jnp.float32),
            pltpu.SemaphoreType.DMA,
        ],
    )
    def k(table_hbm, idx_hbm, out_hbm, idx_v, rows_v, sem):
        w = lax.axis_index("s") * NC + lax.axis_index("c")

        @pl.loop(0, T)
        def _chunks(t):
            cid = t * NW + w

            @pl.when(cid < n_chunks)
            def _():
                pltpu.sync_copy(idx_hbm.at[cid], idx_v)
                pltpu.async_copy(table_hbm.at[idx_v], rows_v, sem).wait()
                pltpu.sync_copy(rows_v, out_hbm.at[pl.ds(cid * CB, CB)])

    return k


def _sc_scatter(n_chunks, np_rows, dim):
    T = -(-n_chunks // NW)
    rpt = np_rows // NS
    mesh = plsc.VectorSubcoreMesh(core_axis_name="c", subcore_axis_name="s")

    @functools.partial(
        pl.kernel,
        out_type=jax.ShapeDtypeStruct((NC * np_rows, dim), jnp.float32),
        mesh=mesh,
        scratch_types=[
            pltpu.VMEM((CB,), jnp.int32),
            pltpu.VMEM((CB, dim), jnp.float32),
            pltpu.VMEM((rpt, dim), jnp.float32),
            pltpu.VMEM_SHARED((np_rows, dim), jnp.float32),
            pltpu.SemaphoreType.DMA,
        ],
    )
    def k(vals_hbm, idx_hbm, out_hbm, idx_v, val_v, stage_v, acc_sh, sem):
        c = lax.axis_index("c")
        s = lax.axis_index("s")
        w = s * NC + c

        @pl.loop(0, rpt)
        def _zero(i):
            for j in range(dim // 16):
                stage_v[i, pl.ds(j * 16, 16)] = jnp.zeros((16,), jnp.float32)

        pltpu.sync_copy(stage_v, acc_sh.at[pl.ds(s * rpt, rpt)])
        plsc.subcore_barrier()

        @pl.loop(0, T)
        def _chunks(t):
            cid = t * NW + w

            @pl.when(cid < n_chunks)
            def _():
                pltpu.sync_copy(idx_hbm.at[cid], idx_v)
                pltpu.sync_copy(vals_hbm.at[pl.ds(cid * CB, CB)], val_v)
                pltpu.sync_copy(val_v, acc_sh.at[idx_v], add=True)

        plsc.subcore_barrier()
        pltpu.sync_copy(acc_sh.at[pl.ds(s * rpt, rpt)], stage_v)
        pltpu.sync_copy(
            stage_v, out_hbm.at[pl.ds(c * np_rows + s * rpt, rpt)])

    return k


def _tc_tp(e_rows, dim):
    mu = np.linspace(0.0, 8.0, 5)

    def body(rows_ref, r_ref, w_ref, out_ref):
        rows = rows_ref[...]
        rr = r_ref[...]
        z = [rows * jnp.exp(-0.125 * (rr - mu[i]) ** 2) for i in range(5)]
        z.append(rows)
        zc = jnp.concatenate(z, axis=1)
        s = jnp.dot(zc, w_ref[...], preferred_element_type=jnp.float32)
        out_ref[...] = s * jax.nn.sigmoid(s)

    return pl.pallas_call(
        body,
        grid=(e_rows // BE,),
        in_specs=[
            pl.BlockSpec((BE, dim), lambda i: (i, 0)),
            pl.BlockSpec((BE, 1), lambda i: (i, 0)),
            pl.BlockSpec((NF * dim, dim), lambda i: (0, 0)),
        ],
        out_specs=pl.BlockSpec((BE, dim), lambda i: (i, 0)),
        out_shape=jax.ShapeDtypeStruct((e_rows, dim), jnp.float32),
    )


def _tc_update(np_rows, dim):
    def body(y0_r, a0_r, b0_r, y1_r, a1_r, b1_r, o0_r, o1_r):
        o0_r[...] = y0_r[...] + a0_r[...] + b0_r[...]
        o1_r[...] = y1_r[...] + a1_r[...] + b1_r[...]

    bs = pl.BlockSpec((BE, dim), lambda i: (i, 0))
    return pl.pallas_call(
        body,
        grid=(np_rows // BE,),
        in_specs=[bs] * 6,
        out_specs=[bs, bs],
        out_shape=[jax.ShapeDtypeStruct((np_rows, dim), jnp.float32)] * 2,
    )


def _tc_readout(na, dim):
    def body(y0_r, a0_r, b0_r, y1_r, a1_r, b1_r, wro_r, bro_r, out_ref):
        t0 = y0_r[...] + a0_r[...] + b0_r[...]
        t1 = y1_r[...] + a1_r[...] + b1_r[...]
        v = jnp.dot(jnp.concatenate([t0, t1], axis=0), wro_r[...],
                    preferred_element_type=jnp.float32) + bro_r[0, 0]
        ps = jnp.sum(v * jax.nn.sigmoid(v))

        @pl.when(pl.program_id(0) == 0)
        def _():
            out_ref[0, 0] = 0.0

        out_ref[0, 0] += ps

    bs = pl.BlockSpec((BR, dim), lambda i: (i, 0))
    return pl.pallas_call(
        body,
        grid=(na // BR,),
        in_specs=[bs] * 6 + [
            pl.BlockSpec((dim, 1), lambda i: (0, 0)),
            pl.BlockSpec(memory_space=pltpu.SMEM),
        ],
        out_specs=pl.BlockSpec(memory_space=pltpu.SMEM),
        out_shape=jax.ShapeDtypeStruct((1, 1), jnp.float32),
    )


def kernel(z0, z1, src, dst, r, r_hat, edges, natoms0, natoms1,
           W_emb, b_emb, Ws2d, Wd2s, W_ro, b_ro):
    E = src.shape[0]
    dim = W_emb.shape[1]
    na0, na1 = z0.shape[0], z1.shape[0]
    n_layers = Ws2d.shape[0]
    np_rows = -(-max(na0, na1) // CB) * CB

    i32 = jnp.int32
    srcc = src.astype(i32).reshape(E // CB, CB)
    dstc = dst.astype(i32).reshape(E // CB, CB)
    r_col = r.astype(jnp.float32).reshape(E, 1)
    emb = W_emb.astype(jnp.float32) + b_emb[None, :].astype(jnp.float32)
    z0p = jnp.concatenate(
        [z0.astype(i32), jnp.zeros((np_rows - na0,), i32)]).reshape(-1, CB)
    z1p = jnp.concatenate(
        [z1.astype(i32), jnp.zeros((np_rows - na1,), i32)]).reshape(-1, CB)

    scale = (1.0 / np.sqrt(NF * dim)) / jnp.sqrt(
        jnp.float32(natoms0 + natoms1))

    g_emb = _sc_gather(np_rows // CB, dim)
    g_edge = _sc_gather(E // CB, dim)
    scat = _sc_scatter(E // CB, np_rows, dim)
    tp = _tc_tp(E, dim)
    upd = _tc_update(np_rows, dim)
    ro = _tc_readout(na0, dim)

    y0 = g_emb(emb, z0p)
    y1 = g_emb(emb, z1p)

    for l in range(n_layers):
        w_s2d = Ws2d[l].reshape(NF * dim, dim) * scale
        w_d2s = Wd2s[l].reshape(NF * dim, dim) * scale
        rows_s = g_edge(y0, srcc)
        msg_s2d = tp(rows_s, r_col, w_s2d)
        p1 = scat(msg_s2d, dstc)
        rows_d = g_edge(y1, dstc)
        msg_d2s = tp(rows_d, r_col, w_d2s)
        p0 = scat(msg_d2s, srcc)
        if l < n_layers - 1:
            y0, y1 = upd(y0, p0[:np_rows], p0[np_rows:],
                         y1, p1[:np_rows], p1[np_rows:])

    out = ro(y0, p0[:np_rows], p0[np_rows:], y1, p1[:np_rows], p1[np_rows:],
             W_ro.astype(jnp.float32), b_ro.reshape(1, 1).astype(jnp.float32))
    return out.reshape(())
